# Initial kernel scaffold; baseline (speedup 1.0000x reference)
import jax
import jax.numpy as jnp
from jax.experimental import pallas as pl


def kernel(indices, values):
    raise NotImplementedError("write your pallas kernel here")


# speedup vs baseline: 1.4651x; 1.4651x over previous
import functools

import jax
import jax.numpy as jnp
from jax import lax
from jax.experimental import pallas as pl
from jax.experimental.pallas import tpu as pltpu
from jax.experimental.pallas import tpu_sc as plsc

_NC = 2
_NS = 16
_NW = _NC * _NS


def _sc_gather(idx_flat, values):
    total = idx_flat.shape[0]
    assert total % (8 * _NW) == 0
    per_w = total // _NW
    mesh = plsc.VectorSubcoreMesh(core_axis_name="c", subcore_axis_name="s")

    @functools.partial(
        pl.kernel,
        mesh=mesh,
        out_type=jax.ShapeDtypeStruct((total,), jnp.float32),
        scratch_types=[
            pltpu.VMEM((per_w,), jnp.int32),
            pltpu.VMEM((per_w,), jnp.float32),
            pltpu.SemaphoreType.DMA,
        ],
    )
    def k(idx_hbm, values_hbm, out_hbm, idx_v, rows_v, sem):
        wid = lax.axis_index("s") * _NC + lax.axis_index("c")
        base = wid * per_w
        pltpu.sync_copy(idx_hbm.at[pl.ds(base, per_w)], idx_v)
        pltpu.async_copy(values_hbm.at[idx_v], rows_v, sem).wait()
        pltpu.sync_copy(rows_v, out_hbm.at[pl.ds(base, per_w)])

    return k(idx_flat, values)


def kernel(indices, values):
    b, l = indices.shape
    idx_flat = indices.reshape(-1).astype(jnp.int32)
    out = _sc_gather(idx_flat, values)
    return out.reshape(b, l, 1)


# speedup vs baseline: 1.4658x; 1.0005x over previous
import functools

import jax
import jax.numpy as jnp
from jax import lax
from jax.experimental import pallas as pl
from jax.experimental.pallas import tpu as pltpu
from jax.experimental.pallas import tpu_sc as plsc

_NC = 2
_NS = 16
_NW = _NC * _NS
_NSTREAM = 4


def _sc_gather(idx_flat, values):
    total = idx_flat.shape[0]
    assert total % (8 * _NW * _NSTREAM) == 0
    per_w = total // _NW
    chunk = per_w // _NSTREAM
    mesh = plsc.VectorSubcoreMesh(core_axis_name="c", subcore_axis_name="s")

    @functools.partial(
        pl.kernel,
        mesh=mesh,
        out_type=jax.ShapeDtypeStruct((total,), jnp.float32),
        scratch_types=[
            pltpu.VMEM((per_w,), jnp.int32),
            pltpu.VMEM((per_w,), jnp.float32),
            pltpu.SemaphoreType.DMA,
        ],
    )
    def k(idx_hbm, values_hbm, out_hbm, idx_v, rows_v, sem):
        wid = lax.axis_index("s") * _NC + lax.axis_index("c")
        base = wid * per_w
        pltpu.sync_copy(idx_hbm.at[pl.ds(base, per_w)], idx_v)
        cps = []
        for j in range(_NSTREAM):
            cps.append(pltpu.async_copy(
                values_hbm.at[idx_v.at[pl.ds(j * chunk, chunk)]],
                rows_v.at[pl.ds(j * chunk, chunk)], sem))
        for cp in cps:
            cp.wait()
        pltpu.sync_copy(rows_v, out_hbm.at[pl.ds(base, per_w)])

    return k(idx_flat, values)


def kernel(indices, values):
    b, l = indices.shape
    idx_flat = indices.reshape(-1).astype(jnp.int32)
    out = _sc_gather(idx_flat, values)
    return out.reshape(b, l, 1)


# speedup vs baseline: 1.7749x; 1.2109x over previous
import functools

import jax
import jax.numpy as jnp
from jax import lax
from jax.experimental import pallas as pl
from jax.experimental.pallas import tpu as pltpu
from jax.experimental.pallas import tpu_sc as plsc

_NC = 2
_NS = 16
_NW = _NC * _NS
_STAGERS = 10
_STAGE_TOTAL = 100_000
_STAGE_ROUND = 25_000


def _sc_gather(idx_flat, values):
    total = idx_flat.shape[0]
    nvals = values.shape[0]
    assert total % (8 * _NW) == 0
    assert nvals == _STAGERS * _STAGE_TOTAL
    per_w = total // _NW
    nrounds = _STAGE_TOTAL // _STAGE_ROUND
    mesh = plsc.VectorSubcoreMesh(core_axis_name="c", subcore_axis_name="s")

    @functools.partial(
        pl.kernel,
        mesh=mesh,
        out_type=jax.ShapeDtypeStruct((total,), jnp.float32),
        scratch_types=[
            pltpu.VMEM_SHARED((nvals,), jnp.float32),
            pltpu.VMEM((per_w,), jnp.int32),
            pltpu.VMEM((per_w,), jnp.float32),
            pltpu.SemaphoreType.DMA,
            pltpu.SemaphoreType.DMA,
        ],
    )
    def k(idx_hbm, values_hbm, out_hbm, shared, idx_v, rows_v,
          sem, isem):
        c = lax.axis_index("c")
        s = lax.axis_index("s")
        wid = s * _NC + c
        base = wid * per_w
        idx_cp = pltpu.async_copy(idx_hbm.at[pl.ds(base, per_w)], idx_v, isem)

        @pl.when(s < _STAGERS)
        def _stage():
            bounce = rows_v.at[pl.ds(0, _STAGE_ROUND)]
            for j in range(nrounds):
                off = s * _STAGE_TOTAL + j * _STAGE_ROUND
                pltpu.sync_copy(values_hbm.at[pl.ds(off, _STAGE_ROUND)],
                                bounce)
                pltpu.sync_copy(bounce, shared.at[pl.ds(off, _STAGE_ROUND)])

        plsc.subcore_barrier()
        idx_cp.wait()
        pltpu.async_copy(shared.at[idx_v], rows_v, sem).wait()
        pltpu.sync_copy(rows_v, out_hbm.at[pl.ds(base, per_w)])

    return k(idx_flat, values)


def kernel(indices, values):
    b, l = indices.shape
    idx_flat = indices.reshape(-1).astype(jnp.int32)
    out = _sc_gather(idx_flat, values)
    return out.reshape(b, l, 1)
